# baseline (speedup 1.0000x reference)
import functools

import jax
import jax.numpy as jnp
from jax import lax
from jax.experimental import pallas as pl
from jax.experimental.pallas import tpu as pltpu, tpu_sc as plsc

B, L = 4096, 200
TOK_D = 64
POS_D = 64
D = TOK_D + POS_D
C = 128
NC, NS = 2, 16
NW = NC * NS
LQ, LR = L // 8, 8
NBUF = 4


def _native_tile_view(x):
    return x.T.reshape(LQ, LR, B // C, C).transpose(0, 2, 1, 3)


def _sc_embed(tok4, pos4, token_table, pos_table):
    mesh = plsc.VectorSubcoreMesh(core_axis_name="c", subcore_axis_name="s")

    scratch = (
        [pltpu.VMEM((LQ, LR, C), jnp.int32)] * 2
        + [pltpu.VMEM((C, TOK_D), jnp.float32)] * NBUF
        + [pltpu.VMEM((C, POS_D), jnp.float32)] * NBUF
        + [pltpu.SemaphoreType.DMA] * (2 * NBUF)
        + [pltpu.VMEM_SHARED((2048, POS_D), jnp.float32)]
    )

    @functools.partial(
        pl.kernel,
        mesh=mesh,
        out_type=jax.ShapeDtypeStruct((B, L, D), jnp.float32),
        compiler_params=pltpu.CompilerParams(use_tc_tiling_on_sc=False),
        scratch_types=scratch,
    )
    def k(tok_hbm, pos_hbm, ttab_hbm, ptab_hbm, out_hbm, tok_idx, pos_idx,
          *bufs):
        tok_rows = bufs[0:NBUF]
        pos_rows = bufs[NBUF:2 * NBUF]
        sem_g = bufs[2 * NBUF:3 * NBUF]
        sem_w = bufs[3 * NBUF:4 * NBUF]
        ptab_sh = bufs[4 * NBUF]

        wid = lax.axis_index("s") * NC + lax.axis_index("c")
        b0 = wid * C

        sid = lax.axis_index("s")
        pltpu.sync_copy(ptab_hbm.at[pl.ds(sid * 128, 128)], pos_rows[0])
        pltpu.sync_copy(pos_rows[0], ptab_sh.at[pl.ds(sid * 128, 128)])

        pltpu.sync_copy(tok_hbm.at[:, wid], tok_idx)
        pltpu.sync_copy(pos_hbm.at[:, wid], pos_idx)
        plsc.subcore_barrier()

        def issue_gather(j, b):
            pltpu.async_copy(ttab_hbm.at[tok_idx.at[j // LR, j % LR]],
                             tok_rows[b], sem_g[b])

        def wait_gather(b, j):
            pltpu.sync_copy(ptab_sh.at[pos_idx.at[j // LR, j % LR]],
                            pos_rows[b])
            pltpu.make_async_copy(ttab_hbm.at[tok_idx.at[0, 0]], tok_rows[b],
                                  sem_g[b]).wait()

        def issue_write(j, b):
            pltpu.async_copy(
                tok_rows[b], out_hbm.at[pl.ds(b0, C), j, pl.ds(0, TOK_D)],
                sem_w[b])
            pltpu.async_copy(
                pos_rows[b], out_hbm.at[pl.ds(b0, C), j, pl.ds(TOK_D, POS_D)],
                sem_w[b])

        def wait_write(j, b):
            pltpu.make_async_copy(
                tok_rows[b], out_hbm.at[pl.ds(b0, C), j, pl.ds(0, TOK_D)],
                sem_w[b]).wait()
            pltpu.make_async_copy(
                pos_rows[b], out_hbm.at[pl.ds(b0, C), j, pl.ds(TOK_D, POS_D)],
                sem_w[b]).wait()

        DL = 3

        def substep(j, phase, do_wait_w, do_prefetch):
            bp = (phase + DL) % NBUF
            if do_wait_w:
                wait_write(j + DL - NBUF, bp)
            if do_prefetch:
                issue_gather(j + DL, bp)
            wait_gather(phase, j)
            issue_write(j, phase)

        for j in range(DL):
            issue_gather(j, j % NBUF)

        for j in range(NBUF):
            substep(j, j, do_wait_w=(j + DL - NBUF >= 0), do_prefetch=True)

        @pl.loop(NBUF, L - NBUF, step=NBUF)
        def _(j0):
            for b in range(NBUF):
                substep(j0 + b, b, do_wait_w=True, do_prefetch=True)

        for b in range(NBUF):
            j = L - NBUF + b
            substep(j, b, do_wait_w=(j + DL < L), do_prefetch=(j + DL < L))

        for b in range(NBUF):
            wait_write(L - NBUF + b, (L - NBUF + b) % NBUF)

    return k(tok4, pos4, token_table, pos_table)


def kernel(tokens, pos, token_table, pos_table):
    tok4 = _native_tile_view(tokens)
    pos4 = _native_tile_view(pos)
    return _sc_embed(tok4, pos4, token_table, pos_table)


# speedup vs baseline: 1.0041x; 1.0041x over previous
import functools

import jax
import jax.numpy as jnp
from jax import lax
from jax.experimental import pallas as pl
from jax.experimental.pallas import tpu as pltpu, tpu_sc as plsc

B, L = 4096, 200
TOK_D = 64
POS_D = 64
D = TOK_D + POS_D
C = 128
NC, NS = 2, 16
NW = NC * NS
LQ, LR = L // 8, 8
NBUF = 4


def _native_tile_view(x):
    return x.T.reshape(LQ, LR, B // C, C).transpose(0, 2, 1, 3)


def _sc_embed(tok4, pos4, token_table, pos_table):
    mesh = plsc.VectorSubcoreMesh(core_axis_name="c", subcore_axis_name="s")

    scratch = (
        [pltpu.VMEM((LQ, LR, C), jnp.int32)] * 2
        + [pltpu.VMEM((C, TOK_D), jnp.float32)] * NBUF
        + [pltpu.VMEM((C, POS_D), jnp.float32)] * NBUF
        + [pltpu.SemaphoreType.DMA] * (2 * NBUF)
        + [pltpu.VMEM_SHARED((2048, POS_D), jnp.float32)]
    )

    @functools.partial(
        pl.kernel,
        mesh=mesh,
        out_type=jax.ShapeDtypeStruct((B, L, D), jnp.float32),
        compiler_params=pltpu.CompilerParams(use_tc_tiling_on_sc=False),
        scratch_types=scratch,
    )
    def k(tok_hbm, pos_hbm, ttab_hbm, ptab_hbm, out_hbm, tok_idx, pos_idx,
          *bufs):
        tok_rows = bufs[0:NBUF]
        pos_rows = bufs[NBUF:2 * NBUF]
        sem_g = bufs[2 * NBUF:3 * NBUF]
        sem_w = bufs[3 * NBUF:4 * NBUF]
        ptab_sh = bufs[4 * NBUF]

        wid = lax.axis_index("s") * NC + lax.axis_index("c")
        b0 = wid * C

        sid = lax.axis_index("s")
        pltpu.sync_copy(ptab_hbm.at[pl.ds(sid * 128, 128)], pos_rows[0])
        pltpu.sync_copy(pos_rows[0], ptab_sh.at[pl.ds(sid * 128, 128)])

        pltpu.sync_copy(tok_hbm.at[:, wid], tok_idx)
        pltpu.sync_copy(pos_hbm.at[:, wid], pos_idx)
        plsc.subcore_barrier()

        def issue_gather(j, b):
            pltpu.async_copy(ttab_hbm.at[tok_idx.at[j // LR, j % LR]],
                             tok_rows[b], sem_g[b])

        def wait_gather(b, j):
            pltpu.sync_copy(ptab_sh.at[pos_idx.at[j // LR, j % LR]],
                            pos_rows[b])
            pltpu.make_async_copy(ttab_hbm.at[tok_idx.at[0, 0]], tok_rows[b],
                                  sem_g[b]).wait()

        def issue_write(j, b):
            pltpu.async_copy(
                tok_rows[b], out_hbm.at[pl.ds(b0, C), j, pl.ds(0, TOK_D)],
                sem_w[b])
            pltpu.async_copy(
                pos_rows[b], out_hbm.at[pl.ds(b0, C), j, pl.ds(TOK_D, POS_D)],
                sem_w[b])

        def wait_write(j, b):
            pltpu.make_async_copy(
                tok_rows[b], out_hbm.at[pl.ds(b0, C), j, pl.ds(0, TOK_D)],
                sem_w[b]).wait()
            pltpu.make_async_copy(
                pos_rows[b], out_hbm.at[pl.ds(b0, C), j, pl.ds(TOK_D, POS_D)],
                sem_w[b]).wait()

        DL = NBUF // 2

        def substep(j, phase, do_wait_w, do_prefetch):
            bp = (phase + DL) % NBUF
            if do_wait_w:
                wait_write(j + DL - NBUF, bp)
            if do_prefetch:
                issue_gather(j + DL, bp)
            wait_gather(phase, j)
            issue_write(j, phase)

        for j in range(DL):
            issue_gather(j, j % NBUF)

        for j in range(NBUF):
            substep(j, j, do_wait_w=(j + DL - NBUF >= 0), do_prefetch=True)

        @pl.loop(NBUF, L - NBUF, step=NBUF)
        def _(j0):
            for b in range(NBUF):
                substep(j0 + b, b, do_wait_w=True, do_prefetch=True)

        for b in range(NBUF):
            j = L - NBUF + b
            substep(j, b, do_wait_w=(j + DL < L), do_prefetch=(j + DL < L))

        for b in range(NBUF):
            wait_write(L - NBUF + b, (L - NBUF + b) % NBUF)

    return k(tok4, pos4, token_table, pos_table)


def kernel(tokens, pos, token_table, pos_table):
    tok4 = _native_tile_view(tokens)
    pos4 = _native_tile_view(pos)
    return _sc_embed(tok4, pos4, token_table, pos_table)
